# Initial kernel scaffold; baseline (speedup 1.0000x reference)
import jax
import jax.numpy as jnp
from jax.experimental import pallas as pl


def kernel(x, edge_index, W_l1, W_r1, b1, W_l2, W_r2, b2):
    raise NotImplementedError("write your pallas kernel here")


# speedup vs baseline: 4.1795x; 4.1795x over previous
import functools

import jax
import jax.numpy as jnp
from jax import lax
from jax.experimental import pallas as pl
from jax.experimental.pallas import tpu as pltpu
from jax.experimental.pallas import tpu_sc as plsc

N_NODES = 10000
N_EDGES = 320000
IN_DIM = 128
HIDDEN = 64

NP = 10240
DEAD = 10200
NC = 2
NS = 16
NW = NC * NS
K = 128
C = 80
ROWS_PER_TILE = NP // NS
RB = 1280
W1 = HIDDEN + 16
W2 = HIDDEN


def _sc_segment_sum(width):
  @functools.partial(
      pl.kernel,
      out_type=jax.ShapeDtypeStruct((NC, NP, width), jnp.float32),
      mesh=plsc.VectorSubcoreMesh(core_axis_name="c", subcore_axis_name="s"),
      scratch_types=[
          pltpu.VMEM((C, K), jnp.int32),
          pltpu.VMEM((C, K), jnp.int32),
          pltpu.VMEM((K, width), jnp.float32),
          pltpu.VMEM_SHARED((NP, width), jnp.float32),
          pltpu.SemaphoreType.DMA,
      ],
      compiler_params=pltpu.CompilerParams(use_tc_tiling_on_sc=False),
  )
  def sc_kernel(z_hbm, src_hbm, dst_hbm, zero_hbm, out_hbm,
                src_v, dst_v, rows_v, acc, sem):
    c = lax.axis_index("c")
    s = lax.axis_index("s")
    w = c * NS + s
    r0 = s * ROWS_PER_TILE
    pltpu.sync_copy(zero_hbm.at[pl.ds(r0, ROWS_PER_TILE)],
                    acc.at[pl.ds(r0, ROWS_PER_TILE)])
    pltpu.sync_copy(src_hbm.at[w], src_v)
    pltpu.sync_copy(dst_hbm.at[w], dst_v)
    plsc.subcore_barrier()

    def body(j, carry):
      pltpu.async_copy(z_hbm.at[src_v.at[j]], rows_v, sem).wait()
      pltpu.sync_copy(rows_v, acc.at[dst_v.at[j]], add=True)
      return carry

    lax.fori_loop(0, C, body, 0)
    plsc.subcore_barrier()
    pltpu.sync_copy(acc.at[pl.ds(r0, ROWS_PER_TILE)],
                    out_hbm.at[c, pl.ds(r0, ROWS_PER_TILE)])

  return sc_kernel


def _tc1(xp, W_l1, W_r1):
  def body(x_ref, wl_ref, wr_ref, z_ref, xl_ref):
    x = x_ref[...]
    zr = jnp.dot(x, wr_ref[...], preferred_element_type=jnp.float32)
    extra = (lax.broadcasted_iota(jnp.int32, (RB, W1 - HIDDEN), 1) == 0)
    z_ref[...] = jnp.concatenate([zr, extra.astype(jnp.float32)], axis=1)
    xl_ref[...] = jnp.dot(x, wl_ref[...], preferred_element_type=jnp.float32)

  return pl.pallas_call(
      body,
      grid=(NP // RB,),
      in_specs=[
          pl.BlockSpec((RB, IN_DIM), lambda i: (i, 0)),
          pl.BlockSpec((IN_DIM, HIDDEN), lambda i: (0, 0)),
          pl.BlockSpec((IN_DIM, HIDDEN), lambda i: (0, 0)),
      ],
      out_specs=[
          pl.BlockSpec((RB, W1), lambda i: (i, 0)),
          pl.BlockSpec((RB, HIDDEN), lambda i: (i, 0)),
      ],
      out_shape=[
          jax.ShapeDtypeStruct((NP, W1), jnp.float32),
          jax.ShapeDtypeStruct((NP, HIDDEN), jnp.float32),
      ],
  )(xp, W_l1, W_r1)


def _tc2(xl1, P1, b1, W_l2, W_r2):
  def body(xl_ref, p_ref, b_ref, wl_ref, wr_ref, z2_ref, hl2_ref, rdeg_ref):
    p = p_ref[0] + p_ref[1]
    s1 = p[:, 0:HIDDEN]
    deg = jnp.maximum(p[:, HIDDEN:HIDDEN + 1], 1.0)
    h1 = jnp.maximum(xl_ref[...] + s1 / deg + b_ref[...], 0.0)
    z2_ref[...] = jnp.dot(h1, wr_ref[...], preferred_element_type=jnp.float32)
    hl2_ref[...] = jnp.dot(h1, wl_ref[...], preferred_element_type=jnp.float32)
    rdeg_ref[...] = jnp.broadcast_to(1.0 / deg, (RB, HIDDEN))

  return pl.pallas_call(
      body,
      grid=(NP // RB,),
      in_specs=[
          pl.BlockSpec((RB, HIDDEN), lambda i: (i, 0)),
          pl.BlockSpec((NC, RB, W1), lambda i: (0, i, 0)),
          pl.BlockSpec((1, HIDDEN), lambda i: (0, 0)),
          pl.BlockSpec((HIDDEN, HIDDEN), lambda i: (0, 0)),
          pl.BlockSpec((HIDDEN, HIDDEN), lambda i: (0, 0)),
      ],
      out_specs=[
          pl.BlockSpec((RB, HIDDEN), lambda i: (i, 0)),
          pl.BlockSpec((RB, HIDDEN), lambda i: (i, 0)),
          pl.BlockSpec((RB, HIDDEN), lambda i: (i, 0)),
      ],
      out_shape=[
          jax.ShapeDtypeStruct((NP, HIDDEN), jnp.float32),
          jax.ShapeDtypeStruct((NP, HIDDEN), jnp.float32),
          jax.ShapeDtypeStruct((NP, HIDDEN), jnp.float32),
      ],
  )(xl1, P1, b1, W_l2, W_r2)


def _tc3(hl2, P2, rdegb, b2):
  def body(hl_ref, p_ref, r_ref, b_ref, out_ref):
    s2 = p_ref[0] + p_ref[1]
    out_ref[...] = hl_ref[...] + s2 * r_ref[...] + b_ref[...]

  return pl.pallas_call(
      body,
      grid=(NP // RB,),
      in_specs=[
          pl.BlockSpec((RB, HIDDEN), lambda i: (i, 0)),
          pl.BlockSpec((NC, RB, W2), lambda i: (0, i, 0)),
          pl.BlockSpec((RB, HIDDEN), lambda i: (i, 0)),
          pl.BlockSpec((1, HIDDEN), lambda i: (0, 0)),
      ],
      out_specs=pl.BlockSpec((RB, HIDDEN), lambda i: (i, 0)),
      out_shape=jax.ShapeDtypeStruct((NP, HIDDEN), jnp.float32),
  )(hl2, P2, rdegb, b2)


@jax.jit
def kernel(x, edge_index, W_l1, W_r1, b1, W_l2, W_r2, b2):
  x = x.astype(jnp.float32)
  src = edge_index[0].astype(jnp.int32)
  dst = edge_index[1].astype(jnp.int32)
  pad = NW * C * K - N_EDGES
  srcs = jnp.concatenate([src, jnp.zeros((pad,), jnp.int32)]).reshape(NW, C, K)
  dsts = jnp.concatenate([dst, jnp.full((pad,), DEAD, jnp.int32)]).reshape(NW, C, K)
  xp = jnp.pad(x, ((0, NP - N_NODES), (0, 0)))

  z1p, xl1 = _tc1(xp, W_l1, W_r1)
  P1 = _sc_segment_sum(W1)(z1p, srcs, dsts, jnp.zeros((NP, W1), jnp.float32))
  z2, hl2, rdegb = _tc2(xl1, P1, b1.reshape(1, HIDDEN), W_l2, W_r2)
  P2 = _sc_segment_sum(W2)(z2, srcs, dsts, jnp.zeros((NP, W2), jnp.float32))
  h2 = _tc3(hl2, P2, rdegb, b2.reshape(1, HIDDEN))
  return h2[None, :N_NODES, :]
